# baseline (speedup 1.0000x reference)
import functools

import jax
import jax.numpy as jnp
from jax import lax
from jax.experimental import pallas as pl
from jax.experimental.pallas import tpu as pltpu
from jax.experimental.pallas import tpu_sc as plsc

_NC = 2
_NS = 16
_NW = _NC * _NS

_CHUNK = 32
_NBUF = 2


def kernel(position_ids, table):
    B, S = position_ids.shape
    V, H = table.shape
    R = B * S
    assert R % _NW == 0
    rows_per_w = R // _NW
    assert rows_per_w % (_CHUNK * _NBUF) == 0
    n_chunks = rows_per_w // _CHUNK
    n_groups = n_chunks // _NBUF

    idx = jnp.swapaxes(position_ids.astype(jnp.int32), 0, 1)
    idx = idx.reshape(_NW, n_chunks, _CHUNK)

    mesh = plsc.VectorSubcoreMesh(
        core_axis_name="c", subcore_axis_name="s",
        num_cores=_NC, num_subcores=_NS)

    @functools.partial(
        pl.kernel,
        out_type=jax.ShapeDtypeStruct((S, B, H), jnp.float32),
        mesh=mesh,
        scratch_types=[
            pltpu.VMEM((n_chunks, _CHUNK), jnp.int32),
            pltpu.VMEM((_NBUF, _CHUNK, H), jnp.float32),
        ] + [pltpu.SemaphoreType.DMA] * (2 * _NBUF),
    )
    def _emb(table_hbm, idx_hbm, out3_hbm, idx_v, rows_v, *sems):
        out_hbm = out3_hbm.reshape(R, H)
        gsem = sems[:_NBUF]
        osem = sems[_NBUF:]
        wid = lax.axis_index("s") * _NC + lax.axis_index("c")
        base = wid * rows_per_w
        pltpu.sync_copy(idx_hbm.at[wid], idx_v)

        def start_g(j, b):
            pltpu.async_copy(table_hbm.at[idx_v.at[j]], rows_v.at[b], gsem[b])

        def wait_g(b):
            pltpu.make_async_copy(
                table_hbm.at[idx_v.at[0]], rows_v.at[b], gsem[b]).wait()

        def start_o(j, b):
            pltpu.async_copy(
                rows_v.at[b],
                out_hbm.at[pl.ds(base + j * _CHUNK, _CHUNK)], osem[b])

        def wait_o(b):
            pltpu.make_async_copy(
                rows_v.at[b],
                out_hbm.at[pl.ds(base, _CHUNK)], osem[b]).wait()

        for b in range(_NBUF):
            start_g(b, b)

        def group(g, carry):
            j0 = g * _NBUF
            for b in range(_NBUF):
                wait_g(b)
                start_o(j0 + b, b)
            for b in range(_NBUF):
                wait_o(b)
                start_g(j0 + b + _NBUF, b)
            return carry

        lax.fori_loop(0, n_groups - 1, group, 0)

        j0 = (n_groups - 1) * _NBUF
        for b in range(_NBUF):
            wait_g(b)
            start_o(j0 + b, b)
        for b in range(_NBUF):
            wait_o(b)

    return _emb(table, idx)


# speedup vs baseline: 1.0095x; 1.0095x over previous
import functools

import jax
import jax.numpy as jnp
from jax import lax
from jax.experimental import pallas as pl
from jax.experimental.pallas import tpu as pltpu
from jax.experimental.pallas import tpu_sc as plsc

_NC = 2
_NS = 16
_NW = _NC * _NS

_CHUNK = 16
_NBUF = 4


def kernel(position_ids, table):
    B, S = position_ids.shape
    V, H = table.shape
    R = B * S
    assert R % _NW == 0
    rows_per_w = R // _NW
    assert rows_per_w % (_CHUNK * _NBUF) == 0
    n_chunks = rows_per_w // _CHUNK
    n_groups = n_chunks // _NBUF

    idx = jnp.swapaxes(position_ids.astype(jnp.int32), 0, 1)
    idx = idx.reshape(_NW, n_chunks, _CHUNK)

    mesh = plsc.VectorSubcoreMesh(
        core_axis_name="c", subcore_axis_name="s",
        num_cores=_NC, num_subcores=_NS)

    @functools.partial(
        pl.kernel,
        out_type=jax.ShapeDtypeStruct((S, B, H), jnp.float32),
        mesh=mesh,
        scratch_types=[
            pltpu.VMEM((n_chunks, _CHUNK), jnp.int32),
            pltpu.VMEM((_NBUF, _CHUNK, H), jnp.float32),
        ] + [pltpu.SemaphoreType.DMA] * (2 * _NBUF),
    )
    def _emb(table_hbm, idx_hbm, out3_hbm, idx_v, rows_v, *sems):
        out_hbm = out3_hbm.reshape(R, H)
        gsem = sems[:_NBUF]
        osem = sems[_NBUF:]
        wid = lax.axis_index("s") * _NC + lax.axis_index("c")
        base = wid * rows_per_w
        pltpu.sync_copy(idx_hbm.at[wid], idx_v)

        def start_g(j, b):
            pltpu.async_copy(table_hbm.at[idx_v.at[j]], rows_v.at[b], gsem[b])

        def wait_g(b):
            pltpu.make_async_copy(
                table_hbm.at[idx_v.at[0]], rows_v.at[b], gsem[b]).wait()

        def start_o(j, b):
            pltpu.async_copy(
                rows_v.at[b],
                out_hbm.at[pl.ds(base + j * _CHUNK, _CHUNK)], osem[b])

        def wait_o(b):
            pltpu.make_async_copy(
                rows_v.at[b],
                out_hbm.at[pl.ds(base, _CHUNK)], osem[b]).wait()

        for b in range(_NBUF):
            start_g(b, b)

        def group(g, carry):
            j0 = g * _NBUF
            for b in range(_NBUF):
                wait_g(b)
                start_o(j0 + b, b)
            for b in range(_NBUF):
                wait_o(b)
                start_g(j0 + b + _NBUF, b)
            return carry

        lax.fori_loop(0, n_groups - 1, group, 0)

        j0 = (n_groups - 1) * _NBUF
        for b in range(_NBUF):
            wait_g(b)
            start_o(j0 + b, b)
        for b in range(_NBUF):
            wait_o(b)

    return _emb(table, idx)


# speedup vs baseline: 1.0116x; 1.0020x over previous
import functools

import jax
import jax.numpy as jnp
from jax import lax
from jax.experimental import pallas as pl
from jax.experimental.pallas import tpu as pltpu
from jax.experimental.pallas import tpu_sc as plsc

_NC = 2
_NS = 16
_NW = _NC * _NS

_CHUNK = 16
_NBUF = 4


def kernel(position_ids, table):
    B, S = position_ids.shape
    V, H = table.shape
    n_blk = _NW // B
    s_per_w = S // n_blk
    assert S % n_blk == 0
    assert s_per_w % (_CHUNK * _NBUF) == 0
    n_chunks = s_per_w // _CHUNK
    n_groups = n_chunks // _NBUF

    idx = position_ids.astype(jnp.int32).reshape(B, n_blk, n_chunks, _CHUNK)

    mesh = plsc.VectorSubcoreMesh(
        core_axis_name="c", subcore_axis_name="s",
        num_cores=_NC, num_subcores=_NS)

    @functools.partial(
        pl.kernel,
        out_type=jax.ShapeDtypeStruct((S, B, H), jnp.float32),
        mesh=mesh,
        scratch_types=[
            pltpu.VMEM((n_chunks, _CHUNK), jnp.int32),
            pltpu.VMEM((_NBUF, _CHUNK, H), jnp.float32),
        ] + [pltpu.SemaphoreType.DMA] * (2 * _NBUF),
    )
    def _emb(table_hbm, idx_hbm, out_hbm, idx_v, rows_v, *sems):
        gsem = sems[:_NBUF]
        osem = sems[_NBUF:]
        wid = lax.axis_index("s") * _NC + lax.axis_index("c")
        b = wid % B
        blk = wid // B
        s_base = blk * s_per_w
        pltpu.sync_copy(idx_hbm.at[b, blk], idx_v)

        def start_g(j, buf):
            pltpu.async_copy(
                table_hbm.at[idx_v.at[j]], rows_v.at[buf], gsem[buf])

        def wait_g(buf):
            pltpu.make_async_copy(
                table_hbm.at[idx_v.at[0]], rows_v.at[buf], gsem[buf]).wait()

        def start_o(j, buf):
            pltpu.async_copy(
                rows_v.at[buf],
                out_hbm.at[pl.ds(s_base + j * _CHUNK, _CHUNK), b], osem[buf])

        def wait_o(buf):
            pltpu.make_async_copy(
                rows_v.at[buf],
                out_hbm.at[pl.ds(s_base, _CHUNK), b], osem[buf]).wait()

        for buf in range(_NBUF):
            start_g(buf, buf)

        def group(g, carry):
            j0 = g * _NBUF
            for buf in range(_NBUF):
                wait_g(buf)
                start_o(j0 + buf, buf)
            for buf in range(_NBUF):
                wait_o(buf)
                start_g(j0 + buf + _NBUF, buf)
            return carry

        lax.fori_loop(0, n_groups - 1, group, 0)

        j0 = (n_groups - 1) * _NBUF
        for buf in range(_NBUF):
            wait_g(buf)
            start_o(j0 + buf, buf)
        for buf in range(_NBUF):
            wait_o(buf)

    return _emb(table, idx)
